# baseline (speedup 1.0000x reference)
import dataclasses
import functools

import jax
import jax.numpy as jnp
from jax import lax
from jax.experimental import pallas as pl
from jax.experimental.pallas import tpu as pltpu
from jax.experimental.pallas import tpu_sc as plsc

NC = 2
NS = 16
LANES = 16
TILE = 128
HALF = 136
POSC = 256 - HALF
ONEC = POSC + 3
DISTC = ONEC + 1


def _sc_segment_sums(src, dst, aug0, aug1, px, py, pz, zrow, n_pad, n_tiles):
    mesh = plsc.VectorSubcoreMesh(core_axis_name="c", subcore_axis_name="s")
    cp = pltpu.CompilerParams()
    if "needs_layout_passes" in pltpu.CompilerParams.__dataclass_fields__:
        cp = dataclasses.replace(cp, needs_layout_passes=False)
    if "use_tc_tiling_on_sc" in pltpu.CompilerParams.__dataclass_fields__:
        cp = dataclasses.replace(cp, use_tc_tiling_on_sc=False)
    nblocks = n_pad // TILE

    bufset = [
        pltpu.VMEM((TILE,), jnp.int32),
        pltpu.VMEM((TILE,), jnp.int32),
        pltpu.VMEM((TILE, HALF), jnp.float32),
        pltpu.VMEM((TILE,), jnp.float32),
        pltpu.VMEM((TILE,), jnp.float32),
        pltpu.VMEM((TILE,), jnp.float32),
        pltpu.VMEM((TILE,), jnp.float32),
        pltpu.VMEM((TILE,), jnp.float32),
        pltpu.VMEM((TILE,), jnp.float32),
        pltpu.SemaphoreType.DMA,
        pltpu.SemaphoreType.DMA,
    ]

    @functools.partial(
        pl.kernel,
        out_type=[jax.ShapeDtypeStruct((n_pad, HALF), jnp.float32),
                  jax.ShapeDtypeStruct((n_pad, HALF), jnp.float32)],
        mesh=mesh,
        compiler_params=cp,
        scratch_types=bufset + bufset + [
            pltpu.VMEM_SHARED((n_pad, HALF), jnp.float32),
        ],
    )
    def body(src_hbm, dst_hbm, aug0_hbm, aug1_hbm, px_hbm, py_hbm, pz_hbm,
             zrow_hbm, out0_hbm, out1_hbm, *refs):
        bufs = (refs[0:11], refs[11:22])
        acc = refs[22]
        cid = lax.axis_index("c")
        sid = lax.axis_index("s")
        nk = (n_tiles - sid + NS - 1) // NS

        def idx_copies(k, buf):
            t = (sid + k * NS) * TILE
            srcv, dstv = buf[0], buf[1]
            return [pltpu.make_async_copy(src_hbm.at[pl.ds(t, TILE)], srcv,
                                          buf[10]),
                    pltpu.make_async_copy(dst_hbm.at[pl.ds(t, TILE)], dstv,
                                          buf[10])]

        def gather_copies(buf):
            srcv, dstv, rows = buf[0], buf[1], buf[2]
            if_sem = buf[9]
            g0 = [pltpu.make_async_copy(aug0_hbm.at[srcv], rows, if_sem)]
            g1 = [pltpu.make_async_copy(aug1_hbm.at[srcv], rows, if_sem),
                  pltpu.make_async_copy(px_hbm.at[srcv], buf[3], if_sem),
                  pltpu.make_async_copy(py_hbm.at[srcv], buf[4], if_sem),
                  pltpu.make_async_copy(pz_hbm.at[srcv], buf[5], if_sem),
                  pltpu.make_async_copy(px_hbm.at[dstv], buf[6], if_sem),
                  pltpu.make_async_copy(py_hbm.at[dstv], buf[7], if_sem),
                  pltpu.make_async_copy(pz_hbm.at[dstv], buf[8], if_sem)]
            return g0, g1

        def start_gathers(buf):
            g0, g1 = gather_copies(buf)

            @pl.when(cid == 0)
            def _():
                for c in g0:
                    c.start()

            @pl.when(cid == 1)
            def _():
                for c in g1:
                    c.start()

        def wait_gathers(buf):
            g0, g1 = gather_copies(buf)

            @pl.when(cid == 0)
            def _():
                for c in g0:
                    c.wait()

            @pl.when(cid == 1)
            def _():
                for c in g1:
                    c.wait()

        def compute_dist(buf):
            rows = buf[2]

            @pl.when(cid == 1)
            def _():
                @pl.loop(0, TILE // LANES)
                def _(i):
                    sl = pl.ds(i * LANES, LANES)
                    dx = buf[6][sl] - buf[3][sl]
                    dy = buf[7][sl] - buf[4][sl]
                    dz = buf[8][sl] - buf[5][sl]
                    d2 = dx * dx + dy * dy + dz * dz
                    d2c = jnp.maximum(d2, 1e-30)
                    bits = plsc.bitcast(d2c, jnp.int32)
                    y = plsc.bitcast(jnp.int32(0x5F3759DF) - (bits >> 1),
                                     jnp.float32)
                    y = y * (1.5 - 0.5 * d2c * y * y)
                    y = y * (1.5 - 0.5 * d2c * y * y)
                    y = y * (1.5 - 0.5 * d2c * y * y)
                    dist = d2 * y
                    rowid = lax.iota(jnp.int32, LANES) + i * LANES
                    colid = jnp.full((LANES,), DISTC, jnp.int32)
                    plsc.store_scatter(rows, [rowid, colid], dist)

        def process(k, cur, nxt):
            @pl.when(k + 1 < nk)
            def _():
                for c in idx_copies(k + 1, nxt):
                    c.wait()
                start_gathers(nxt)
            wait_gathers(cur)
            compute_dist(cur)
            pltpu.sync_copy(cur[2], acc.at[cur[1]], add=True)

            @pl.when(k + 2 < nk)
            def _():
                for c in idx_copies(k + 2, cur):
                    c.start()

        pltpu.sync_copy(zrow_hbm, bufs[0][2])

        @pl.loop(sid, nblocks, step=NS)
        def _(b):
            pltpu.sync_copy(bufs[0][2], acc.at[pl.ds(b * TILE, TILE)])

        plsc.subcore_barrier()

        @pl.when(nk > 0)
        def _():
            for c in idx_copies(0, bufs[0]):
                c.start()
                c.wait()
            start_gathers(bufs[0])

            @pl.when(1 < nk)
            def _():
                for c in idx_copies(1, bufs[1]):
                    c.start()

        @pl.loop(0, (nk + 1) // 2)
        def _(p):
            process(2 * p, bufs[0], bufs[1])

            @pl.when(2 * p + 1 < nk)
            def _():
                process(2 * p + 1, bufs[1], bufs[0])

        plsc.subcore_barrier()

        @pl.when(cid == 0)
        def _():
            @pl.loop(sid, nblocks, step=NS)
            def _(b):
                pltpu.sync_copy(acc.at[pl.ds(b * TILE, TILE)],
                                out0_hbm.at[pl.ds(b * TILE, TILE)])

        @pl.when(cid == 1)
        def _():
            @pl.loop(sid, nblocks, step=NS)
            def _(b):
                pltpu.sync_copy(acc.at[pl.ds(b * TILE, TILE)],
                                out1_hbm.at[pl.ds(b * TILE, TILE)])

    return body(src, dst, aug0, aug1, px, py, pz, zrow)


def _tc_matmul(a, w, block_m):
    m, k = a.shape
    _, n = w.shape

    def mm(a_ref, w_ref, o_ref):
        o_ref[...] = jnp.dot(a_ref[...], w_ref[...],
                             preferred_element_type=jnp.float32,
                             precision=lax.Precision.HIGHEST)

    return pl.pallas_call(
        mm,
        grid=(m // block_m,),
        in_specs=[pl.BlockSpec((block_m, k), lambda i: (i, 0)),
                  pl.BlockSpec((k, n), lambda i: (0, 0))],
        out_specs=pl.BlockSpec((block_m, n), lambda i: (i, 0)),
        out_shape=jax.ShapeDtypeStruct((m, n), jnp.float32),
    )(a, w)


def kernel(input_feature, pos, edge_index, W_neighbor, b_neighbor, W_self,
           b_self):
    n, d_in = input_feature.shape
    e = edge_index.shape[1]
    d_out = W_self.shape[1]
    assert e % TILE == 0
    n_tiles = e // TILE
    n_pad = ((n + TILE - 1) // TILE) * TILE

    feat = input_feature.astype(jnp.float32)
    pos = pos.astype(jnp.float32)
    src = edge_index[0].astype(jnp.int32)
    dst = edge_index[1].astype(jnp.int32)

    aug0 = feat[:, :HALF]
    aug1 = jnp.concatenate(
        [feat[:, HALF:], pos, jnp.ones((n, 1), jnp.float32),
         jnp.zeros((n, HALF - DISTC), jnp.float32)], axis=1)
    px = pos[:, 0] + 0.0
    py = pos[:, 1] + 0.0
    pz = pos[:, 2] + 0.0
    zrow = jnp.zeros((TILE, HALF), jnp.float32)

    acc0, acc1 = _sc_segment_sums(src, dst, aug0, aug1, px, py, pz, zrow,
                                  n_pad, n_tiles)

    feat_sum = jnp.concatenate([acc0[:n], acc1[:n, :POSC]], axis=1)
    possum = acc1[:n, POSC:POSC + 3]
    deg = acc1[:n, ONEC:ONEC + 1]
    distsum = acc1[:n, DISTC:DISTC + 1]

    a = jnp.concatenate(
        [feat_sum, deg * feat, deg * pos - possum, distsum, deg,
         jnp.zeros((n, 3), jnp.float32)], axis=1)
    block_m = 1280
    m_pad = ((n + block_m - 1) // block_m) * block_m
    a = jnp.pad(a, ((0, m_pad - n), (0, 0)))
    w_big = jnp.concatenate(
        [W_neighbor[:d_in], W_self, W_neighbor[d_in:d_in + 3],
         W_neighbor[d_in + 3:d_in + 4], (b_neighbor + b_self)[None],
         jnp.zeros((3, d_out), jnp.float32)], axis=0)

    out = _tc_matmul(a, w_big, block_m=block_m)
    return out[:n]


# speedup vs baseline: 1.0572x; 1.0572x over previous
import dataclasses
import functools

import jax
import jax.numpy as jnp
from jax import lax
from jax.experimental import pallas as pl
from jax.experimental.pallas import tpu as pltpu
from jax.experimental.pallas import tpu_sc as plsc

NC = 2
NS = 16
LANES = 16
TILE = 128
HALF = 136
POSC = 256 - HALF
ONEC = POSC + 3
DISTC = ONEC + 1


def _sc_segment_sums(src, dst, aug0, aug1, pos4, zrow, n_pad, n_tiles):
    mesh = plsc.VectorSubcoreMesh(core_axis_name="c", subcore_axis_name="s")
    cp = pltpu.CompilerParams()
    if "needs_layout_passes" in pltpu.CompilerParams.__dataclass_fields__:
        cp = dataclasses.replace(cp, needs_layout_passes=False)
    if "use_tc_tiling_on_sc" in pltpu.CompilerParams.__dataclass_fields__:
        cp = dataclasses.replace(cp, use_tc_tiling_on_sc=False)
    nblocks = n_pad // TILE

    bufset = [
        pltpu.VMEM((TILE,), jnp.int32),
        pltpu.VMEM((TILE,), jnp.int32),
        pltpu.VMEM((TILE, HALF), jnp.float32),
        pltpu.VMEM((TILE, 8), jnp.float32),
        pltpu.VMEM((TILE, 8), jnp.float32),
        pltpu.SemaphoreType.DMA,
        pltpu.SemaphoreType.DMA,
    ]

    @functools.partial(
        pl.kernel,
        out_type=[jax.ShapeDtypeStruct((n_pad, HALF), jnp.float32),
                  jax.ShapeDtypeStruct((n_pad, HALF), jnp.float32)],
        mesh=mesh,
        compiler_params=cp,
        scratch_types=bufset + bufset + [
            pltpu.VMEM_SHARED((n_pad, HALF), jnp.float32),
        ],
    )
    def body(src_hbm, dst_hbm, aug0_hbm, aug1_hbm, pos4_hbm,
             zrow_hbm, out0_hbm, out1_hbm, *refs):
        bufs = (refs[0:7], refs[7:14])
        acc = refs[14]
        cid = lax.axis_index("c")
        sid = lax.axis_index("s")
        nk = (n_tiles - sid + NS - 1) // NS

        def idx_copies(k, buf):
            t = (sid + k * NS) * TILE
            srcv, dstv = buf[0], buf[1]
            return [pltpu.make_async_copy(src_hbm.at[pl.ds(t, TILE)], srcv,
                                          buf[6]),
                    pltpu.make_async_copy(dst_hbm.at[pl.ds(t, TILE)], dstv,
                                          buf[6])]

        def gather_copies(buf):
            srcv, dstv, rows = buf[0], buf[1], buf[2]
            if_sem = buf[5]
            g0 = [pltpu.make_async_copy(aug0_hbm.at[srcv], rows, if_sem)]
            g1 = [pltpu.make_async_copy(aug1_hbm.at[srcv], rows, if_sem),
                  pltpu.make_async_copy(pos4_hbm.at[srcv], buf[3], if_sem),
                  pltpu.make_async_copy(pos4_hbm.at[dstv], buf[4], if_sem)]
            return g0, g1

        def start_gathers(buf):
            g0, g1 = gather_copies(buf)

            @pl.when(cid == 0)
            def _():
                for c in g0:
                    c.start()

            @pl.when(cid == 1)
            def _():
                for c in g1:
                    c.start()

        def wait_gathers(buf):
            g0, g1 = gather_copies(buf)

            @pl.when(cid == 0)
            def _():
                for c in g0:
                    c.wait()

            @pl.when(cid == 1)
            def _():
                for c in g1:
                    c.wait()

        def compute_dist(buf):
            rows = buf[2]

            @pl.when(cid == 1)
            def _():
                @pl.loop(0, TILE // LANES)
                def _(i):
                    rowid = lax.iota(jnp.int32, LANES) + i * LANES
                    c0 = jnp.full((LANES,), 0, jnp.int32)
                    c1 = jnp.full((LANES,), 1, jnp.int32)
                    c2 = jnp.full((LANES,), 2, jnp.int32)
                    ps4, pd4 = buf[3], buf[4]
                    dx = (plsc.load_gather(pd4, [rowid, c0])
                          - plsc.load_gather(ps4, [rowid, c0]))
                    dy = (plsc.load_gather(pd4, [rowid, c1])
                          - plsc.load_gather(ps4, [rowid, c1]))
                    dz = (plsc.load_gather(pd4, [rowid, c2])
                          - plsc.load_gather(ps4, [rowid, c2]))
                    d2 = dx * dx + dy * dy + dz * dz
                    d2c = jnp.maximum(d2, 1e-30)
                    bits = plsc.bitcast(d2c, jnp.int32)
                    y = plsc.bitcast(jnp.int32(0x5F3759DF) - (bits >> 1),
                                     jnp.float32)
                    y = y * (1.5 - 0.5 * d2c * y * y)
                    y = y * (1.5 - 0.5 * d2c * y * y)
                    y = y * (1.5 - 0.5 * d2c * y * y)
                    dist = d2 * y
                    colid = jnp.full((LANES,), DISTC, jnp.int32)
                    plsc.store_scatter(rows, [rowid, colid], dist)

        def process(k, cur, nxt):
            @pl.when(k + 1 < nk)
            def _():
                for c in idx_copies(k + 1, nxt):
                    c.wait()
                start_gathers(nxt)
            wait_gathers(cur)
            compute_dist(cur)
            pltpu.sync_copy(cur[2], acc.at[cur[1]], add=True)

            @pl.when(k + 2 < nk)
            def _():
                for c in idx_copies(k + 2, cur):
                    c.start()

        pltpu.sync_copy(zrow_hbm, bufs[0][2])

        @pl.loop(sid, nblocks, step=NS)
        def _(b):
            pltpu.sync_copy(bufs[0][2], acc.at[pl.ds(b * TILE, TILE)])

        plsc.subcore_barrier()

        @pl.when(nk > 0)
        def _():
            for c in idx_copies(0, bufs[0]):
                c.start()
                c.wait()
            start_gathers(bufs[0])

            @pl.when(1 < nk)
            def _():
                for c in idx_copies(1, bufs[1]):
                    c.start()

        @pl.loop(0, (nk + 1) // 2)
        def _(p):
            process(2 * p, bufs[0], bufs[1])

            @pl.when(2 * p + 1 < nk)
            def _():
                process(2 * p + 1, bufs[1], bufs[0])

        plsc.subcore_barrier()

        @pl.when(cid == 0)
        def _():
            @pl.loop(sid, nblocks, step=NS)
            def _(b):
                pltpu.sync_copy(acc.at[pl.ds(b * TILE, TILE)],
                                out0_hbm.at[pl.ds(b * TILE, TILE)])

        @pl.when(cid == 1)
        def _():
            @pl.loop(sid, nblocks, step=NS)
            def _(b):
                pltpu.sync_copy(acc.at[pl.ds(b * TILE, TILE)],
                                out1_hbm.at[pl.ds(b * TILE, TILE)])

    return body(src, dst, aug0, aug1, pos4, zrow)


def _tc_matmul(a, w, block_m):
    m, k = a.shape
    _, n = w.shape

    def mm(a_ref, w_ref, o_ref):
        o_ref[...] = jnp.dot(a_ref[...], w_ref[...],
                             preferred_element_type=jnp.float32,
                             precision=lax.Precision.HIGHEST)

    return pl.pallas_call(
        mm,
        grid=(m // block_m,),
        in_specs=[pl.BlockSpec((block_m, k), lambda i: (i, 0)),
                  pl.BlockSpec((k, n), lambda i: (0, 0))],
        out_specs=pl.BlockSpec((block_m, n), lambda i: (i, 0)),
        out_shape=jax.ShapeDtypeStruct((m, n), jnp.float32),
    )(a, w)


def kernel(input_feature, pos, edge_index, W_neighbor, b_neighbor, W_self,
           b_self):
    n, d_in = input_feature.shape
    e = edge_index.shape[1]
    d_out = W_self.shape[1]
    assert e % TILE == 0
    n_tiles = e // TILE
    n_pad = ((n + TILE - 1) // TILE) * TILE

    feat = input_feature.astype(jnp.float32)
    pos = pos.astype(jnp.float32)
    src = edge_index[0].astype(jnp.int32)
    dst = edge_index[1].astype(jnp.int32)

    aug0 = feat[:, :HALF]
    aug1 = jnp.concatenate(
        [feat[:, HALF:], pos, jnp.ones((n, 1), jnp.float32),
         jnp.zeros((n, HALF - DISTC), jnp.float32)], axis=1)
    pos4 = jnp.concatenate([pos, jnp.zeros((n, 5), jnp.float32)], axis=1)
    zrow = jnp.zeros((TILE, HALF), jnp.float32)

    acc0, acc1 = _sc_segment_sums(src, dst, aug0, aug1, pos4, zrow,
                                  n_pad, n_tiles)

    feat_sum = jnp.concatenate([acc0[:n], acc1[:n, :POSC]], axis=1)
    possum = acc1[:n, POSC:POSC + 3]
    deg = acc1[:n, ONEC:ONEC + 1]
    distsum = acc1[:n, DISTC:DISTC + 1]

    a = jnp.concatenate(
        [feat_sum, deg * feat, deg * pos - possum, distsum, deg,
         jnp.zeros((n, 3), jnp.float32)], axis=1)
    block_m = 1280
    m_pad = ((n + block_m - 1) // block_m) * block_m
    a = jnp.pad(a, ((0, m_pad - n), (0, 0)))
    w_big = jnp.concatenate(
        [W_neighbor[:d_in], W_self, W_neighbor[d_in:d_in + 3],
         W_neighbor[d_in + 3:d_in + 4], (b_neighbor + b_self)[None],
         jnp.zeros((3, d_out), jnp.float32)], axis=0)

    out = _tc_matmul(a, w_big, block_m=block_m)
    return out[:n]


# speedup vs baseline: 1.4281x; 1.3509x over previous
import dataclasses
import functools

import jax
import jax.numpy as jnp
from jax import lax
from jax.experimental import pallas as pl
from jax.experimental.pallas import tpu as pltpu
from jax.experimental.pallas import tpu_sc as plsc

NC = 2
NS = 16
LANES = 16
TILE = 128
HALF = 136
POSC = 256 - HALF
ONEC = POSC + 3
DISTC = ONEC + 1


def _sc_segment_sums(src, dst, aug0, aug1, pos4, zrow, n_pad, n_tiles):
    mesh = plsc.VectorSubcoreMesh(core_axis_name="c", subcore_axis_name="s")
    cp = pltpu.CompilerParams()
    if "needs_layout_passes" in pltpu.CompilerParams.__dataclass_fields__:
        cp = dataclasses.replace(cp, needs_layout_passes=False)
    if "use_tc_tiling_on_sc" in pltpu.CompilerParams.__dataclass_fields__:
        cp = dataclasses.replace(cp, use_tc_tiling_on_sc=False)
    nblocks = n_pad // TILE

    bufset = [
        pltpu.VMEM((TILE,), jnp.int32),
        pltpu.VMEM((TILE,), jnp.int32),
        pltpu.VMEM((TILE, HALF), jnp.float32),
        pltpu.VMEM((TILE, 8), jnp.float32),
        pltpu.VMEM((TILE, 8), jnp.float32),
        pltpu.SemaphoreType.DMA,
        pltpu.SemaphoreType.DMA,
    ]

    @functools.partial(
        pl.kernel,
        out_type=[jax.ShapeDtypeStruct((n_pad, HALF), jnp.float32),
                  jax.ShapeDtypeStruct((n_pad, HALF), jnp.float32)],
        mesh=mesh,
        compiler_params=cp,
        scratch_types=bufset + bufset + [
            pltpu.VMEM_SHARED((n_pad, HALF), jnp.float32),
        ],
    )
    def body(src_hbm, dst_hbm, aug0_hbm, aug1_hbm, pos4_hbm,
             zrow_hbm, out0_hbm, out1_hbm, *refs):
        bufs = (refs[0:7], refs[7:14])
        acc = refs[14]
        cid = lax.axis_index("c")
        sid = lax.axis_index("s")
        nk = (n_tiles - sid + NS - 1) // NS

        def idx_copies(k, buf):
            t = (sid + k * NS) * TILE
            srcv, dstv = buf[0], buf[1]
            return [pltpu.make_async_copy(src_hbm.at[pl.ds(t, TILE)], srcv,
                                          buf[6]),
                    pltpu.make_async_copy(dst_hbm.at[pl.ds(t, TILE)], dstv,
                                          buf[6])]

        def gather_copies(buf):
            srcv, dstv, rows = buf[0], buf[1], buf[2]
            if_sem = buf[5]
            g0 = [pltpu.make_async_copy(aug0_hbm.at[srcv], rows, if_sem)]
            g1 = [pltpu.make_async_copy(aug1_hbm.at[srcv], rows, if_sem),
                  pltpu.make_async_copy(pos4_hbm.at[srcv], buf[3], if_sem),
                  pltpu.make_async_copy(pos4_hbm.at[dstv], buf[4], if_sem)]
            return g0, g1

        def start_gathers(buf):
            g0, g1 = gather_copies(buf)

            @pl.when(cid == 0)
            def _():
                for c in g0:
                    c.start()

            @pl.when(cid == 1)
            def _():
                for c in g1:
                    c.start()

        def wait_gathers(buf):
            g0, g1 = gather_copies(buf)

            @pl.when(cid == 0)
            def _():
                for c in g0:
                    c.wait()

            @pl.when(cid == 1)
            def _():
                for c in g1:
                    c.wait()

        def compute_dist(buf):
            rows = buf[2]

            @pl.when(cid == 1)
            def _():
                @pl.loop(0, TILE // LANES)
                def _(i):
                    rowid = lax.iota(jnp.int32, LANES) + i * LANES
                    c0 = jnp.full((LANES,), 0, jnp.int32)
                    c1 = jnp.full((LANES,), 1, jnp.int32)
                    c2 = jnp.full((LANES,), 2, jnp.int32)
                    ps4, pd4 = buf[3], buf[4]
                    dx = (plsc.load_gather(pd4, [rowid, c0])
                          - plsc.load_gather(ps4, [rowid, c0]))
                    dy = (plsc.load_gather(pd4, [rowid, c1])
                          - plsc.load_gather(ps4, [rowid, c1]))
                    dz = (plsc.load_gather(pd4, [rowid, c2])
                          - plsc.load_gather(ps4, [rowid, c2]))
                    d2 = dx * dx + dy * dy + dz * dz
                    d2c = jnp.maximum(d2, 1e-30)
                    bits = plsc.bitcast(d2c, jnp.int32)
                    y = plsc.bitcast(jnp.int32(0x5F3759DF) - (bits >> 1),
                                     jnp.float32)
                    y = y * (1.5 - 0.5 * d2c * y * y)
                    y = y * (1.5 - 0.5 * d2c * y * y)
                    y = y * (1.5 - 0.5 * d2c * y * y)
                    dist = d2 * y
                    colid = jnp.full((LANES,), DISTC, jnp.int32)
                    plsc.store_scatter(rows, [rowid, colid], dist)

        def process(k, cur, nxt):
            @pl.when(k + 1 < nk)
            def _():
                for c in idx_copies(k + 1, nxt):
                    c.wait()
                start_gathers(nxt)
            wait_gathers(cur)
            compute_dist(cur)
            pltpu.sync_copy(cur[2], acc.at[cur[1]], add=True)

            @pl.when(k + 2 < nk)
            def _():
                for c in idx_copies(k + 2, cur):
                    c.start()

        pltpu.sync_copy(zrow_hbm, bufs[0][2])

        @pl.loop(sid, nblocks, step=NS)
        def _(b):
            pltpu.sync_copy(bufs[0][2], acc.at[pl.ds(b * TILE, TILE)])

        plsc.subcore_barrier()

        @pl.when(nk > 0)
        def _():
            for c in idx_copies(0, bufs[0]):
                c.start()
                c.wait()
            start_gathers(bufs[0])

            @pl.when(1 < nk)
            def _():
                for c in idx_copies(1, bufs[1]):
                    c.start()

        @pl.loop(0, (nk + 1) // 2)
        def _(p):
            process(2 * p, bufs[0], bufs[1])

            @pl.when(2 * p + 1 < nk)
            def _():
                process(2 * p + 1, bufs[1], bufs[0])

        plsc.subcore_barrier()

        @pl.when(cid == 0)
        def _():
            @pl.loop(sid, nblocks, step=NS)
            def _(b):
                pltpu.sync_copy(acc.at[pl.ds(b * TILE, TILE)],
                                out0_hbm.at[pl.ds(b * TILE, TILE)])

        @pl.when(cid == 1)
        def _():
            @pl.loop(sid, nblocks, step=NS)
            def _(b):
                pltpu.sync_copy(acc.at[pl.ds(b * TILE, TILE)],
                                out1_hbm.at[pl.ds(b * TILE, TILE)])

    return body(src, dst, aug0, aug1, pos4, zrow)


def _tc_fused_out(acc0, acc1, feat, pos, w0, w1, ws, small, n, d_out):
    block_m = 1000
    hi = lax.Precision.HIGHEST

    def mm(a0_ref, a1_ref, f_ref, p_ref, w0_ref, w1_ref, ws_ref, s_ref,
           o_ref):
        a1 = a1_ref[...]
        deg = a1[:, ONEC:ONEC + 1]
        s = s_ref[...]
        o = jnp.dot(a0_ref[...], w0_ref[...],
                    preferred_element_type=jnp.float32, precision=hi)
        o += jnp.dot(a1[:, :POSC], w1_ref[...],
                     preferred_element_type=jnp.float32, precision=hi)
        o += jnp.dot(deg * f_ref[...], ws_ref[...],
                     preferred_element_type=jnp.float32, precision=hi)
        p = p_ref[...]
        for c in range(3):
            rel = deg * p[:, c:c + 1] - a1[:, POSC + c:POSC + c + 1]
            o += rel * s[c:c + 1, :]
        o += a1[:, DISTC:DISTC + 1] * s[3:4, :]
        o += deg * s[4:5, :]
        o_ref[...] = o

    return pl.pallas_call(
        mm,
        grid=(n // block_m,),
        in_specs=[pl.BlockSpec((block_m, HALF), lambda i: (i, 0)),
                  pl.BlockSpec((block_m, HALF), lambda i: (i, 0)),
                  pl.BlockSpec((block_m, 256), lambda i: (i, 0)),
                  pl.BlockSpec((block_m, 3), lambda i: (i, 0)),
                  pl.BlockSpec((HALF, 256), lambda i: (0, 0)),
                  pl.BlockSpec((POSC, 256), lambda i: (0, 0)),
                  pl.BlockSpec((256, 256), lambda i: (0, 0)),
                  pl.BlockSpec((8, 256), lambda i: (0, 0))],
        out_specs=pl.BlockSpec((block_m, d_out), lambda i: (i, 0)),
        out_shape=jax.ShapeDtypeStruct((n, d_out), jnp.float32),
    )(acc0, acc1, feat, pos, w0, w1, ws, small)


def kernel(input_feature, pos, edge_index, W_neighbor, b_neighbor, W_self,
           b_self):
    n, d_in = input_feature.shape
    e = edge_index.shape[1]
    d_out = W_self.shape[1]
    assert e % TILE == 0
    n_tiles = e // TILE
    n_pad = ((n + TILE - 1) // TILE) * TILE

    feat = input_feature.astype(jnp.float32)
    pos = pos.astype(jnp.float32)
    src = edge_index[0].astype(jnp.int32)
    dst = edge_index[1].astype(jnp.int32)

    aug0 = feat[:, :HALF]
    aug1 = jnp.concatenate(
        [feat[:, HALF:], pos, jnp.ones((n, 1), jnp.float32),
         jnp.zeros((n, HALF - DISTC), jnp.float32)], axis=1)
    pos4 = jnp.concatenate([pos, jnp.zeros((n, 5), jnp.float32)], axis=1)
    zrow = jnp.zeros((TILE, HALF), jnp.float32)

    acc0, acc1 = _sc_segment_sums(src, dst, aug0, aug1, pos4, zrow,
                                  n_pad, n_tiles)

    w0 = W_neighbor[:HALF]
    w1 = W_neighbor[HALF:d_in]
    small = jnp.concatenate(
        [W_neighbor[d_in:d_in + 4], (b_neighbor + b_self)[None],
         jnp.zeros((3, d_out), jnp.float32)], axis=0)

    return _tc_fused_out(acc0, acc1, feat, pos, w0, w1, W_self, small,
                         n, d_out)


# speedup vs baseline: 1.7330x; 1.2135x over previous
import dataclasses
import functools

import jax
import jax.numpy as jnp
from jax import lax
from jax.experimental import pallas as pl
from jax.experimental.pallas import tpu as pltpu
from jax.experimental.pallas import tpu_sc as plsc

NC = 2
NS = 16
LANES = 16
TILE = 128
HALF = 128
NARROW = 8
NONEC = 3
NDIST = 4


def _sc_segment_sums(src, dst, aug0, aug1, nar, zrow, zrow8, n_pad, n_tiles):
    mesh = plsc.VectorSubcoreMesh(core_axis_name="c", subcore_axis_name="s")
    cp = pltpu.CompilerParams()
    if "needs_layout_passes" in pltpu.CompilerParams.__dataclass_fields__:
        cp = dataclasses.replace(cp, needs_layout_passes=False)
    if "use_tc_tiling_on_sc" in pltpu.CompilerParams.__dataclass_fields__:
        cp = dataclasses.replace(cp, use_tc_tiling_on_sc=False)
    nblocks = n_pad // TILE

    bufset = [
        pltpu.VMEM((TILE,), jnp.int32),
        pltpu.VMEM((TILE,), jnp.int32),
        pltpu.VMEM((TILE, HALF), jnp.float32),
        pltpu.VMEM((TILE, NARROW), jnp.float32),
        pltpu.VMEM((TILE, NARROW), jnp.float32),
        pltpu.SemaphoreType.DMA,
        pltpu.SemaphoreType.DMA,
    ]

    @functools.partial(
        pl.kernel,
        out_type=[jax.ShapeDtypeStruct((n_pad, HALF), jnp.float32),
                  jax.ShapeDtypeStruct((n_pad, HALF), jnp.float32),
                  jax.ShapeDtypeStruct((n_pad, NARROW), jnp.float32),
                  jax.ShapeDtypeStruct((n_pad, NARROW), jnp.float32)],
        mesh=mesh,
        compiler_params=cp,
        scratch_types=bufset + bufset + [
            pltpu.VMEM_SHARED((n_pad, HALF), jnp.float32),
            pltpu.VMEM_SHARED((n_pad, NARROW), jnp.float32),
        ],
    )
    def body(src_hbm, dst_hbm, aug0_hbm, aug1_hbm, nar_hbm, zrow_hbm,
             zrow8_hbm, out0_hbm, out1_hbm, outn0_hbm, outn1_hbm, *refs):
        bufs = (refs[0:7], refs[7:14])
        acc, accn = refs[14], refs[15]
        cid = lax.axis_index("c")
        sid = lax.axis_index("s")
        nk = (n_tiles - sid + NS - 1) // NS

        def idx_copies(k, buf):
            t = (sid + k * NS) * TILE
            return [pltpu.make_async_copy(src_hbm.at[pl.ds(t, TILE)], buf[0],
                                          buf[6]),
                    pltpu.make_async_copy(dst_hbm.at[pl.ds(t, TILE)], buf[1],
                                          buf[6])]

        def narrow_here(k):
            return (k % 2) == cid

        def gather_copies(buf):
            srcv, dstv = buf[0], buf[1]
            g0 = [pltpu.make_async_copy(aug0_hbm.at[srcv], buf[2], buf[5])]
            g1 = [pltpu.make_async_copy(aug1_hbm.at[srcv], buf[2], buf[5])]
            gn = [pltpu.make_async_copy(nar_hbm.at[srcv], buf[3], buf[5]),
                  pltpu.make_async_copy(nar_hbm.at[dstv], buf[4], buf[5])]
            return g0, g1, gn

        def start_gathers(k, buf):
            g0, g1, gn = gather_copies(buf)

            @pl.when(cid == 0)
            def _():
                g0[0].start()

            @pl.when(cid == 1)
            def _():
                g1[0].start()

            @pl.when(narrow_here(k))
            def _():
                for c in gn:
                    c.start()

        def wait_gathers(k, buf):
            g0, g1, gn = gather_copies(buf)

            @pl.when(cid == 0)
            def _():
                g0[0].wait()

            @pl.when(cid == 1)
            def _():
                g1[0].wait()

            @pl.when(narrow_here(k))
            def _():
                for c in gn:
                    c.wait()

        def narrow_work(k, buf):
            @pl.when(narrow_here(k))
            def _():
                ns, nd = buf[3], buf[4]

                @pl.loop(0, TILE // LANES)
                def _(i):
                    rowid = lax.iota(jnp.int32, LANES) + i * LANES
                    c0 = jnp.full((LANES,), 0, jnp.int32)
                    c1 = jnp.full((LANES,), 1, jnp.int32)
                    c2 = jnp.full((LANES,), 2, jnp.int32)
                    dx = (plsc.load_gather(nd, [rowid, c0])
                          - plsc.load_gather(ns, [rowid, c0]))
                    dy = (plsc.load_gather(nd, [rowid, c1])
                          - plsc.load_gather(ns, [rowid, c1]))
                    dz = (plsc.load_gather(nd, [rowid, c2])
                          - plsc.load_gather(ns, [rowid, c2]))
                    d2 = dx * dx + dy * dy + dz * dz
                    d2c = jnp.maximum(d2, 1e-30)
                    bits = plsc.bitcast(d2c, jnp.int32)
                    y = plsc.bitcast(jnp.int32(0x5F3759DF) - (bits >> 1),
                                     jnp.float32)
                    y = y * (1.5 - 0.5 * d2c * y * y)
                    y = y * (1.5 - 0.5 * d2c * y * y)
                    y = y * (1.5 - 0.5 * d2c * y * y)
                    dist = d2 * y
                    cd = jnp.full((LANES,), NDIST, jnp.int32)
                    plsc.store_scatter(ns, [rowid, cd], dist)

                pltpu.sync_copy(ns, accn.at[buf[1]], add=True)

        def process(k, cur, nxt):
            @pl.when(k + 1 < nk)
            def _():
                for c in idx_copies(k + 1, nxt):
                    c.wait()
                start_gathers(k + 1, nxt)
            wait_gathers(k, cur)
            narrow_work(k, cur)
            pltpu.sync_copy(cur[2], acc.at[cur[1]], add=True)

            @pl.when(k + 2 < nk)
            def _():
                for c in idx_copies(k + 2, cur):
                    c.start()

        pltpu.sync_copy(zrow_hbm, bufs[0][2])
        pltpu.sync_copy(zrow8_hbm, bufs[0][3])

        @pl.loop(sid, nblocks, step=NS)
        def _(b):
            pltpu.sync_copy(bufs[0][2], acc.at[pl.ds(b * TILE, TILE)])
            pltpu.sync_copy(bufs[0][3], accn.at[pl.ds(b * TILE, TILE)])

        plsc.subcore_barrier()

        @pl.when(nk > 0)
        def _():
            for c in idx_copies(0, bufs[0]):
                c.start()
                c.wait()
            start_gathers(0, bufs[0])

            @pl.when(1 < nk)
            def _():
                for c in idx_copies(1, bufs[1]):
                    c.start()

        @pl.loop(0, (nk + 1) // 2)
        def _(p):
            process(2 * p, bufs[0], bufs[1])

            @pl.when(2 * p + 1 < nk)
            def _():
                process(2 * p + 1, bufs[1], bufs[0])

        plsc.subcore_barrier()

        @pl.when(cid == 0)
        def _():
            @pl.loop(sid, nblocks, step=NS)
            def _(b):
                pltpu.sync_copy(acc.at[pl.ds(b * TILE, TILE)],
                                out0_hbm.at[pl.ds(b * TILE, TILE)])
                pltpu.sync_copy(accn.at[pl.ds(b * TILE, TILE)],
                                outn0_hbm.at[pl.ds(b * TILE, TILE)])

        @pl.when(cid == 1)
        def _():
            @pl.loop(sid, nblocks, step=NS)
            def _(b):
                pltpu.sync_copy(acc.at[pl.ds(b * TILE, TILE)],
                                out1_hbm.at[pl.ds(b * TILE, TILE)])
                pltpu.sync_copy(accn.at[pl.ds(b * TILE, TILE)],
                                outn1_hbm.at[pl.ds(b * TILE, TILE)])

    return body(src, dst, aug0, aug1, nar, zrow, zrow8)


def _tc_fused_out(acc0, acc1, accn0, accn1, feat, pos, w0, w1, ws, small,
                  n, d_out):
    block_m = 1000
    hi = lax.Precision.HIGHEST

    def mm(a0_ref, a1_ref, n0_ref, n1_ref, f_ref, p_ref, w0_ref, w1_ref,
           ws_ref, s_ref, o_ref):
        nb = n0_ref[...] + n1_ref[...]
        deg = nb[:, NONEC:NONEC + 1]
        s = s_ref[...]
        o = jnp.dot(a0_ref[...], w0_ref[...],
                    preferred_element_type=jnp.float32, precision=hi)
        o += jnp.dot(a1_ref[...], w1_ref[...],
                     preferred_element_type=jnp.float32, precision=hi)
        o += jnp.dot(deg * f_ref[...], ws_ref[...],
                     preferred_element_type=jnp.float32, precision=hi)
        p = p_ref[...]
        for c in range(3):
            rel = deg * p[:, c:c + 1] - nb[:, c:c + 1]
            o += rel * s[c:c + 1, :]
        o += nb[:, NDIST:NDIST + 1] * s[3:4, :]
        o += deg * s[4:5, :]
        o_ref[...] = o

    return pl.pallas_call(
        mm,
        grid=(n // block_m,),
        in_specs=[pl.BlockSpec((block_m, HALF), lambda i: (i, 0)),
                  pl.BlockSpec((block_m, HALF), lambda i: (i, 0)),
                  pl.BlockSpec((block_m, NARROW), lambda i: (i, 0)),
                  pl.BlockSpec((block_m, NARROW), lambda i: (i, 0)),
                  pl.BlockSpec((block_m, 256), lambda i: (i, 0)),
                  pl.BlockSpec((block_m, 3), lambda i: (i, 0)),
                  pl.BlockSpec((HALF, 256), lambda i: (0, 0)),
                  pl.BlockSpec((HALF, 256), lambda i: (0, 0)),
                  pl.BlockSpec((256, 256), lambda i: (0, 0)),
                  pl.BlockSpec((8, 256), lambda i: (0, 0))],
        out_specs=pl.BlockSpec((block_m, d_out), lambda i: (i, 0)),
        out_shape=jax.ShapeDtypeStruct((n, d_out), jnp.float32),
    )(acc0, acc1, accn0, accn1, feat, pos, w0, w1, ws, small)


def kernel(input_feature, pos, edge_index, W_neighbor, b_neighbor, W_self,
           b_self):
    n, d_in = input_feature.shape
    e = edge_index.shape[1]
    d_out = W_self.shape[1]
    assert e % TILE == 0 and d_in == 2 * HALF
    n_tiles = e // TILE
    n_pad = ((n + TILE - 1) // TILE) * TILE

    feat = input_feature.astype(jnp.float32)
    pos = pos.astype(jnp.float32)
    src = edge_index[0].astype(jnp.int32)
    dst = edge_index[1].astype(jnp.int32)

    aug0 = feat[:, :HALF]
    aug1 = feat[:, HALF:]
    nar = jnp.concatenate(
        [pos, jnp.ones((n, 1), jnp.float32),
         jnp.zeros((n, NARROW - NONEC - 1), jnp.float32)], axis=1)
    zrow = jnp.zeros((TILE, HALF), jnp.float32)
    zrow8 = jnp.zeros((TILE, NARROW), jnp.float32)

    acc0, acc1, accn0, accn1 = _sc_segment_sums(
        src, dst, aug0, aug1, nar, zrow, zrow8, n_pad, n_tiles)

    w0 = W_neighbor[:HALF]
    w1 = W_neighbor[HALF:d_in]
    small = jnp.concatenate(
        [W_neighbor[d_in:d_in + 4], (b_neighbor + b_self)[None],
         jnp.zeros((3, d_out), jnp.float32)], axis=0)

    return _tc_fused_out(acc0, acc1, accn0, accn1, feat, pos, w0, w1, W_self,
                         small, n, d_out)


# speedup vs baseline: 1.8113x; 1.0452x over previous
import dataclasses
import functools

import jax
import jax.numpy as jnp
from jax import lax
from jax.experimental import pallas as pl
from jax.experimental.pallas import tpu as pltpu
from jax.experimental.pallas import tpu_sc as plsc

NC = 2
NS = 16
LANES = 16
TILE = 128
HALF = 128
NARROW = 8
NONEC = 3
NDIST = 4


def _sc_segment_sums(src, dst, aug0, aug1, nar, zrow, zrow8, n_pad, n_tiles):
    mesh = plsc.VectorSubcoreMesh(core_axis_name="c", subcore_axis_name="s")
    cp = pltpu.CompilerParams()
    if "needs_layout_passes" in pltpu.CompilerParams.__dataclass_fields__:
        cp = dataclasses.replace(cp, needs_layout_passes=False)
    if "use_tc_tiling_on_sc" in pltpu.CompilerParams.__dataclass_fields__:
        cp = dataclasses.replace(cp, use_tc_tiling_on_sc=False)
    nblocks = n_pad // TILE

    bufset = [
        pltpu.VMEM((TILE,), jnp.int32),
        pltpu.VMEM((TILE,), jnp.int32),
        pltpu.VMEM((TILE, HALF), jnp.float32),
        pltpu.VMEM((TILE, NARROW), jnp.float32),
        pltpu.VMEM((TILE, NARROW), jnp.float32),
        pltpu.SemaphoreType.DMA,
        pltpu.SemaphoreType.DMA,
    ]

    @functools.partial(
        pl.kernel,
        out_type=[jax.ShapeDtypeStruct((n_pad, HALF), jnp.float32),
                  jax.ShapeDtypeStruct((n_pad, HALF), jnp.float32),
                  jax.ShapeDtypeStruct((n_pad, NARROW), jnp.float32),
                  jax.ShapeDtypeStruct((n_pad, NARROW), jnp.float32)],
        mesh=mesh,
        compiler_params=cp,
        scratch_types=bufset + bufset + [
            pltpu.VMEM_SHARED((n_pad, HALF), jnp.float32),
            pltpu.VMEM_SHARED((n_pad, NARROW), jnp.float32),
        ],
    )
    def body(src_hbm, dst_hbm, aug0_hbm, aug1_hbm, nar_hbm, zrow_hbm,
             zrow8_hbm, out0_hbm, out1_hbm, outn0_hbm, outn1_hbm, *refs):
        bufs = (refs[0:7], refs[7:14])
        acc, accn = refs[14], refs[15]
        cid = lax.axis_index("c")
        sid = lax.axis_index("s")
        nk = (n_tiles - sid + NS - 1) // NS

        def idx_copies(k, buf):
            t = (sid + k * NS) * TILE
            return [pltpu.make_async_copy(src_hbm.at[pl.ds(t, TILE)], buf[0],
                                          buf[6]),
                    pltpu.make_async_copy(dst_hbm.at[pl.ds(t, TILE)], buf[1],
                                          buf[6])]

        def narrow_here(k):
            return (k % 2) == cid

        def gather_copies(buf):
            srcv, dstv = buf[0], buf[1]
            g0 = [pltpu.make_async_copy(aug0_hbm.at[srcv], buf[2], buf[5])]
            g1 = [pltpu.make_async_copy(aug1_hbm.at[srcv], buf[2], buf[5])]
            gn = [pltpu.make_async_copy(nar_hbm.at[srcv], buf[3], buf[5]),
                  pltpu.make_async_copy(nar_hbm.at[dstv], buf[4], buf[5])]
            return g0, g1, gn

        def start_gathers(k, buf):
            g0, g1, gn = gather_copies(buf)

            @pl.when(cid == 0)
            def _():
                g0[0].start()

            @pl.when(cid == 1)
            def _():
                g1[0].start()

            @pl.when(narrow_here(k))
            def _():
                for c in gn:
                    c.start()

        def wait_gathers(k, buf):
            g0, g1, gn = gather_copies(buf)

            @pl.when(cid == 0)
            def _():
                g0[0].wait()

            @pl.when(cid == 1)
            def _():
                g1[0].wait()

            @pl.when(narrow_here(k))
            def _():
                for c in gn:
                    c.wait()

        def narrow_work(k, buf):
            @pl.when(narrow_here(k))
            def _():
                ns, nd = buf[3], buf[4]

                @pl.loop(0, TILE // LANES)
                def _(i):
                    rowid = lax.iota(jnp.int32, LANES) + i * LANES
                    c0 = jnp.full((LANES,), 0, jnp.int32)
                    c1 = jnp.full((LANES,), 1, jnp.int32)
                    c2 = jnp.full((LANES,), 2, jnp.int32)
                    dx = (plsc.load_gather(nd, [rowid, c0])
                          - plsc.load_gather(ns, [rowid, c0]))
                    dy = (plsc.load_gather(nd, [rowid, c1])
                          - plsc.load_gather(ns, [rowid, c1]))
                    dz = (plsc.load_gather(nd, [rowid, c2])
                          - plsc.load_gather(ns, [rowid, c2]))
                    d2 = dx * dx + dy * dy + dz * dz
                    d2c = jnp.maximum(d2, 1e-30)
                    bits = plsc.bitcast(d2c, jnp.int32)
                    y = plsc.bitcast(jnp.int32(0x5F3759DF) - (bits >> 1),
                                     jnp.float32)
                    y = y * (1.5 - 0.5 * d2c * y * y)
                    y = y * (1.5 - 0.5 * d2c * y * y)
                    y = y * (1.5 - 0.5 * d2c * y * y)
                    dist = d2 * y
                    cd = jnp.full((LANES,), NDIST, jnp.int32)
                    plsc.store_scatter(ns, [rowid, cd], dist)

                pltpu.sync_copy(ns, accn.at[buf[1]], add=True)

        def process(k, cur, nxt):
            @pl.when(k + 1 < nk)
            def _():
                for c in idx_copies(k + 1, nxt):
                    c.wait()
                start_gathers(k + 1, nxt)
            wait_gathers(k, cur)
            narrow_work(k, cur)
            pltpu.sync_copy(cur[2], acc.at[cur[1]], add=True)

            @pl.when(k + 2 < nk)
            def _():
                for c in idx_copies(k + 2, cur):
                    c.start()

        pltpu.sync_copy(zrow_hbm, bufs[0][2])
        pltpu.sync_copy(zrow8_hbm, bufs[0][3])

        @pl.loop(sid, nblocks, step=NS)
        def _(b):
            pltpu.sync_copy(bufs[0][2], acc.at[pl.ds(b * TILE, TILE)])
            pltpu.sync_copy(bufs[0][3], accn.at[pl.ds(b * TILE, TILE)])

        plsc.subcore_barrier()

        @pl.when(nk > 0)
        def _():
            for c in idx_copies(0, bufs[0]):
                c.start()
                c.wait()
            start_gathers(0, bufs[0])

            @pl.when(1 < nk)
            def _():
                for c in idx_copies(1, bufs[1]):
                    c.start()

        @pl.loop(0, (nk + 1) // 2)
        def _(p):
            process(2 * p, bufs[0], bufs[1])

            @pl.when(2 * p + 1 < nk)
            def _():
                process(2 * p + 1, bufs[1], bufs[0])

        plsc.subcore_barrier()

        @pl.when(cid == 0)
        def _():
            @pl.loop(sid, nblocks, step=NS)
            def _(b):
                pltpu.sync_copy(acc.at[pl.ds(b * TILE, TILE)],
                                out0_hbm.at[pl.ds(b * TILE, TILE)])
                pltpu.sync_copy(accn.at[pl.ds(b * TILE, TILE)],
                                outn0_hbm.at[pl.ds(b * TILE, TILE)])

        @pl.when(cid == 1)
        def _():
            @pl.loop(sid, nblocks, step=NS)
            def _(b):
                pltpu.sync_copy(acc.at[pl.ds(b * TILE, TILE)],
                                out1_hbm.at[pl.ds(b * TILE, TILE)])
                pltpu.sync_copy(accn.at[pl.ds(b * TILE, TILE)],
                                outn1_hbm.at[pl.ds(b * TILE, TILE)])

    return body(src, dst, aug0, aug1, nar, zrow, zrow8)


def _tc_fused_out(acc0, acc1, accn0, accn1, feat, pos, w0, w1, ws, small,
                  n, d_out):
    block_m = 1000
    hi = lax.Precision.DEFAULT

    def mm(a0_ref, a1_ref, n0_ref, n1_ref, f_ref, p_ref, w0_ref, w1_ref,
           ws_ref, s_ref, o_ref):
        nb = n0_ref[...] + n1_ref[...]
        deg = nb[:, NONEC:NONEC + 1]
        s = s_ref[...]
        o = jnp.dot(a0_ref[...], w0_ref[...],
                    preferred_element_type=jnp.float32, precision=hi)
        o += jnp.dot(a1_ref[...], w1_ref[...],
                     preferred_element_type=jnp.float32, precision=hi)
        o += jnp.dot(deg * f_ref[...], ws_ref[...],
                     preferred_element_type=jnp.float32, precision=hi)
        p = p_ref[...]
        for c in range(3):
            rel = deg * p[:, c:c + 1] - nb[:, c:c + 1]
            o += rel * s[c:c + 1, :]
        o += nb[:, NDIST:NDIST + 1] * s[3:4, :]
        o += deg * s[4:5, :]
        o_ref[...] = o

    return pl.pallas_call(
        mm,
        grid=(n // block_m,),
        in_specs=[pl.BlockSpec((block_m, HALF), lambda i: (i, 0)),
                  pl.BlockSpec((block_m, HALF), lambda i: (i, 0)),
                  pl.BlockSpec((block_m, NARROW), lambda i: (i, 0)),
                  pl.BlockSpec((block_m, NARROW), lambda i: (i, 0)),
                  pl.BlockSpec((block_m, 256), lambda i: (i, 0)),
                  pl.BlockSpec((block_m, 3), lambda i: (i, 0)),
                  pl.BlockSpec((HALF, 256), lambda i: (0, 0)),
                  pl.BlockSpec((HALF, 256), lambda i: (0, 0)),
                  pl.BlockSpec((256, 256), lambda i: (0, 0)),
                  pl.BlockSpec((8, 256), lambda i: (0, 0))],
        out_specs=pl.BlockSpec((block_m, d_out), lambda i: (i, 0)),
        out_shape=jax.ShapeDtypeStruct((n, d_out), jnp.float32),
    )(acc0, acc1, accn0, accn1, feat, pos, w0, w1, ws, small)


def kernel(input_feature, pos, edge_index, W_neighbor, b_neighbor, W_self,
           b_self):
    n, d_in = input_feature.shape
    e = edge_index.shape[1]
    d_out = W_self.shape[1]
    assert e % TILE == 0 and d_in == 2 * HALF
    n_tiles = e // TILE
    n_pad = ((n + TILE - 1) // TILE) * TILE

    feat = input_feature.astype(jnp.float32)
    pos = pos.astype(jnp.float32)
    src = edge_index[0].astype(jnp.int32)
    dst = edge_index[1].astype(jnp.int32)

    aug0 = feat[:, :HALF]
    aug1 = feat[:, HALF:]
    nar = jnp.concatenate(
        [pos, jnp.ones((n, 1), jnp.float32),
         jnp.zeros((n, NARROW - NONEC - 1), jnp.float32)], axis=1)
    zrow = jnp.zeros((TILE, HALF), jnp.float32)
    zrow8 = jnp.zeros((TILE, NARROW), jnp.float32)

    acc0, acc1, accn0, accn1 = _sc_segment_sums(
        src, dst, aug0, aug1, nar, zrow, zrow8, n_pad, n_tiles)

    w0 = W_neighbor[:HALF]
    w1 = W_neighbor[HALF:d_in]
    small = jnp.concatenate(
        [W_neighbor[d_in:d_in + 4], (b_neighbor + b_self)[None],
         jnp.zeros((3, d_out), jnp.float32)], axis=0)

    return _tc_fused_out(acc0, acc1, accn0, accn1, feat, pos, w0, w1, W_self,
                         small, n, d_out)


# speedup vs baseline: 1.9227x; 1.0615x over previous
import dataclasses
import functools

import jax
import jax.numpy as jnp
from jax import lax
from jax.experimental import pallas as pl
from jax.experimental.pallas import tpu as pltpu
from jax.experimental.pallas import tpu_sc as plsc

NC = 2
NS = 16
LANES = 16
TILE = 128
HALF = 128
NARROW = 8
NONEC = 3
NDIST = 4


def _sc_segment_sums(src, dst, aug0, aug1, nar, zrow, zrow8, n_pad, n_tiles):
    mesh = plsc.VectorSubcoreMesh(core_axis_name="c", subcore_axis_name="s")
    cp = pltpu.CompilerParams()
    if "needs_layout_passes" in pltpu.CompilerParams.__dataclass_fields__:
        cp = dataclasses.replace(cp, needs_layout_passes=False)
    if "use_tc_tiling_on_sc" in pltpu.CompilerParams.__dataclass_fields__:
        cp = dataclasses.replace(cp, use_tc_tiling_on_sc=False)
    nblocks = n_pad // TILE

    bufset = [
        pltpu.VMEM((TILE,), jnp.int32),
        pltpu.VMEM((TILE,), jnp.int32),
        pltpu.VMEM((TILE, HALF), jnp.float32),
        pltpu.VMEM((TILE, NARROW), jnp.float32),
        pltpu.VMEM((TILE, NARROW), jnp.float32),
        pltpu.SemaphoreType.DMA,
        pltpu.SemaphoreType.DMA,
        pltpu.VMEM((TILE,), jnp.int32),
        pltpu.SemaphoreType.DMA,
    ]

    @functools.partial(
        pl.kernel,
        out_type=[jax.ShapeDtypeStruct((n_pad, HALF), jnp.float32),
                  jax.ShapeDtypeStruct((n_pad, HALF), jnp.float32),
                  jax.ShapeDtypeStruct((n_pad, NARROW), jnp.float32),
                  jax.ShapeDtypeStruct((n_pad, NARROW), jnp.float32)],
        mesh=mesh,
        compiler_params=cp,
        scratch_types=bufset + bufset + [
            pltpu.VMEM_SHARED((n_pad, HALF), jnp.float32),
            pltpu.VMEM_SHARED((n_pad, NARROW), jnp.float32),
        ],
    )
    def body(src_hbm, dst_hbm, aug0_hbm, aug1_hbm, nar_hbm, zrow_hbm,
             zrow8_hbm, out0_hbm, out1_hbm, outn0_hbm, outn1_hbm, *refs):
        bufs = (refs[0:9], refs[9:18])
        acc, accn = refs[18], refs[19]
        cid = lax.axis_index("c")
        sid = lax.axis_index("s")
        nk = (n_tiles - sid + NS - 1) // NS

        def idx_copies(k, buf):
            t = (sid + k * NS) * TILE
            return [pltpu.make_async_copy(src_hbm.at[pl.ds(t, TILE)], buf[0],
                                          buf[6]),
                    pltpu.make_async_copy(dst_hbm.at[pl.ds(t, TILE)], buf[1],
                                          buf[6])]

        def narrow_here(k):
            return (k % 2) == cid

        def gather_copies(buf):
            srcv, dstv = buf[0], buf[1]
            g0 = [pltpu.make_async_copy(aug0_hbm.at[srcv], buf[2], buf[5])]
            g1 = [pltpu.make_async_copy(aug1_hbm.at[srcv], buf[2], buf[5])]
            gn = [pltpu.make_async_copy(nar_hbm.at[srcv], buf[3], buf[5]),
                  pltpu.make_async_copy(nar_hbm.at[dstv], buf[4], buf[5])]
            return g0, g1, gn

        def start_gathers(k, buf):
            g0, g1, gn = gather_copies(buf)

            @pl.when(cid == 0)
            def _():
                g0[0].start()

            @pl.when(cid == 1)
            def _():
                g1[0].start()

            @pl.when(narrow_here(k))
            def _():
                for c in gn:
                    c.start()

        def wait_gathers(k, buf):
            g0, g1, gn = gather_copies(buf)

            @pl.when(cid == 0)
            def _():
                g0[0].wait()

            @pl.when(cid == 1)
            def _():
                g1[0].wait()

            @pl.when(narrow_here(k))
            def _():
                for c in gn:
                    c.wait()

        def narrow_work(k, buf):
            @pl.when(narrow_here(k))
            def _():
                ns, nd = buf[3], buf[4]

                @pl.loop(0, TILE // LANES)
                def _(i):
                    rowid = lax.iota(jnp.int32, LANES) + i * LANES
                    c0 = jnp.full((LANES,), 0, jnp.int32)
                    c1 = jnp.full((LANES,), 1, jnp.int32)
                    c2 = jnp.full((LANES,), 2, jnp.int32)
                    dx = (plsc.load_gather(nd, [rowid, c0])
                          - plsc.load_gather(ns, [rowid, c0]))
                    dy = (plsc.load_gather(nd, [rowid, c1])
                          - plsc.load_gather(ns, [rowid, c1]))
                    dz = (plsc.load_gather(nd, [rowid, c2])
                          - plsc.load_gather(ns, [rowid, c2]))
                    d2 = dx * dx + dy * dy + dz * dz
                    d2c = jnp.maximum(d2, 1e-30)
                    bits = plsc.bitcast(d2c, jnp.int32)
                    y = plsc.bitcast(jnp.int32(0x5F3759DF) - (bits >> 1),
                                     jnp.float32)
                    y = y * (1.5 - 0.5 * d2c * y * y)
                    y = y * (1.5 - 0.5 * d2c * y * y)
                    y = y * (1.5 - 0.5 * d2c * y * y)
                    dist = d2 * y
                    cd = jnp.full((LANES,), NDIST, jnp.int32)
                    plsc.store_scatter(ns, [rowid, cd], dist)

        def start_scatters(k, buf):
            @pl.loop(0, TILE // LANES)
            def _(i):
                sl = pl.ds(i * LANES, LANES)
                buf[7][sl] = buf[1][sl]
            pltpu.async_copy(buf[2], acc.at[buf[7]], buf[8], add=True)

            @pl.when(narrow_here(k))
            def _():
                pltpu.async_copy(buf[3], accn.at[buf[7]], buf[8], add=True)

        def wait_scatters(k, buf):
            pltpu.make_async_copy(buf[2], acc.at[buf[7]], buf[8]).wait()

            @pl.when(narrow_here(k))
            def _():
                pltpu.make_async_copy(buf[3], accn.at[buf[7]], buf[8]).wait()

        def process(k, cur, nxt):
            @pl.when(k + 1 < nk)
            def _():
                @pl.when(k >= 1)
                def _():
                    wait_scatters(k - 1, nxt)
                for c in idx_copies(k + 1, nxt):
                    c.wait()
                start_gathers(k + 1, nxt)
            wait_gathers(k, cur)
            narrow_work(k, cur)
            start_scatters(k, cur)

            @pl.when(k + 2 < nk)
            def _():
                for c in idx_copies(k + 2, cur):
                    c.start()

        pltpu.sync_copy(zrow_hbm, bufs[0][2])
        pltpu.sync_copy(zrow8_hbm, bufs[0][3])

        @pl.loop(sid, nblocks, step=NS)
        def _(b):
            pltpu.sync_copy(bufs[0][2], acc.at[pl.ds(b * TILE, TILE)])
            pltpu.sync_copy(bufs[0][3], accn.at[pl.ds(b * TILE, TILE)])

        plsc.subcore_barrier()

        @pl.when(nk > 0)
        def _():
            for c in idx_copies(0, bufs[0]):
                c.start()
                c.wait()
            start_gathers(0, bufs[0])

            @pl.when(1 < nk)
            def _():
                for c in idx_copies(1, bufs[1]):
                    c.start()

        @pl.loop(0, (nk + 1) // 2)
        def _(p):
            process(2 * p, bufs[0], bufs[1])

            @pl.when(2 * p + 1 < nk)
            def _():
                process(2 * p + 1, bufs[1], bufs[0])

        @pl.when(nk >= 1)
        def _():
            wait_scatters(0, bufs[0])

        @pl.when(nk >= 2)
        def _():
            wait_scatters(1, bufs[1])

        plsc.subcore_barrier()

        @pl.when(cid == 0)
        def _():
            @pl.loop(sid, nblocks, step=NS)
            def _(b):
                pltpu.sync_copy(acc.at[pl.ds(b * TILE, TILE)],
                                out0_hbm.at[pl.ds(b * TILE, TILE)])
                pltpu.sync_copy(accn.at[pl.ds(b * TILE, TILE)],
                                outn0_hbm.at[pl.ds(b * TILE, TILE)])

        @pl.when(cid == 1)
        def _():
            @pl.loop(sid, nblocks, step=NS)
            def _(b):
                pltpu.sync_copy(acc.at[pl.ds(b * TILE, TILE)],
                                out1_hbm.at[pl.ds(b * TILE, TILE)])
                pltpu.sync_copy(accn.at[pl.ds(b * TILE, TILE)],
                                outn1_hbm.at[pl.ds(b * TILE, TILE)])

    return body(src, dst, aug0, aug1, nar, zrow, zrow8)


def _tc_fused_out(acc0, acc1, accn0, accn1, feat, pos, w0, w1, ws, small,
                  n, d_out):
    block_m = 1000
    hi = lax.Precision.DEFAULT

    def mm(a0_ref, a1_ref, n0_ref, n1_ref, f_ref, p_ref, w0_ref, w1_ref,
           ws_ref, s_ref, o_ref):
        nb = n0_ref[...] + n1_ref[...]
        deg = nb[:, NONEC:NONEC + 1]
        s = s_ref[...]
        o = jnp.dot(a0_ref[...], w0_ref[...],
                    preferred_element_type=jnp.float32, precision=hi)
        o += jnp.dot(a1_ref[...], w1_ref[...],
                     preferred_element_type=jnp.float32, precision=hi)
        o += jnp.dot(deg * f_ref[...], ws_ref[...],
                     preferred_element_type=jnp.float32, precision=hi)
        p = p_ref[...]
        for c in range(3):
            rel = deg * p[:, c:c + 1] - nb[:, c:c + 1]
            o += rel * s[c:c + 1, :]
        o += nb[:, NDIST:NDIST + 1] * s[3:4, :]
        o += deg * s[4:5, :]
        o_ref[...] = o

    return pl.pallas_call(
        mm,
        grid=(n // block_m,),
        in_specs=[pl.BlockSpec((block_m, HALF), lambda i: (i, 0)),
                  pl.BlockSpec((block_m, HALF), lambda i: (i, 0)),
                  pl.BlockSpec((block_m, NARROW), lambda i: (i, 0)),
                  pl.BlockSpec((block_m, NARROW), lambda i: (i, 0)),
                  pl.BlockSpec((block_m, 256), lambda i: (i, 0)),
                  pl.BlockSpec((block_m, 3), lambda i: (i, 0)),
                  pl.BlockSpec((HALF, 256), lambda i: (0, 0)),
                  pl.BlockSpec((HALF, 256), lambda i: (0, 0)),
                  pl.BlockSpec((256, 256), lambda i: (0, 0)),
                  pl.BlockSpec((8, 256), lambda i: (0, 0))],
        out_specs=pl.BlockSpec((block_m, d_out), lambda i: (i, 0)),
        out_shape=jax.ShapeDtypeStruct((n, d_out), jnp.float32),
    )(acc0, acc1, accn0, accn1, feat, pos, w0, w1, ws, small)


def kernel(input_feature, pos, edge_index, W_neighbor, b_neighbor, W_self,
           b_self):
    n, d_in = input_feature.shape
    e = edge_index.shape[1]
    d_out = W_self.shape[1]
    assert e % TILE == 0 and d_in == 2 * HALF
    n_tiles = e // TILE
    n_pad = ((n + TILE - 1) // TILE) * TILE

    feat = input_feature.astype(jnp.float32)
    pos = pos.astype(jnp.float32)
    src = edge_index[0].astype(jnp.int32)
    dst = edge_index[1].astype(jnp.int32)

    aug0 = feat[:, :HALF]
    aug1 = feat[:, HALF:]
    nar = jnp.concatenate(
        [pos, jnp.ones((n, 1), jnp.float32),
         jnp.zeros((n, NARROW - NONEC - 1), jnp.float32)], axis=1)
    zrow = jnp.zeros((TILE, HALF), jnp.float32)
    zrow8 = jnp.zeros((TILE, NARROW), jnp.float32)

    acc0, acc1, accn0, accn1 = _sc_segment_sums(
        src, dst, aug0, aug1, nar, zrow, zrow8, n_pad, n_tiles)

    w0 = W_neighbor[:HALF]
    w1 = W_neighbor[HALF:d_in]
    small = jnp.concatenate(
        [W_neighbor[d_in:d_in + 4], (b_neighbor + b_self)[None],
         jnp.zeros((3, d_out), jnp.float32)], axis=0)

    return _tc_fused_out(acc0, acc1, accn0, accn1, feat, pos, w0, w1, W_self,
                         small, n, d_out)


# speedup vs baseline: 1.9472x; 1.0128x over previous
import dataclasses
import functools

import jax
import jax.numpy as jnp
from jax import lax
from jax.experimental import pallas as pl
from jax.experimental.pallas import tpu as pltpu
from jax.experimental.pallas import tpu_sc as plsc

NC = 2
NS = 16
LANES = 16
TILE = 128
HALF = 128
NARROW = 8
NONEC = 3
NDIST = 4


def _sc_segment_sums(src, dst, aug0, aug1, nar, zrow, zrow8, n_pad, n_tiles):
    mesh = plsc.VectorSubcoreMesh(core_axis_name="c", subcore_axis_name="s")
    cp = pltpu.CompilerParams()
    if "needs_layout_passes" in pltpu.CompilerParams.__dataclass_fields__:
        cp = dataclasses.replace(cp, needs_layout_passes=False)
    if "use_tc_tiling_on_sc" in pltpu.CompilerParams.__dataclass_fields__:
        cp = dataclasses.replace(cp, use_tc_tiling_on_sc=False)
    nblocks = n_pad // TILE

    bufset = [
        pltpu.VMEM((TILE,), jnp.int32),
        pltpu.VMEM((TILE,), jnp.int32),
        pltpu.VMEM((TILE, HALF), jnp.float32),
        pltpu.VMEM((TILE, NARROW), jnp.float32),
        pltpu.VMEM((TILE, NARROW), jnp.float32),
        pltpu.SemaphoreType.DMA,
        pltpu.SemaphoreType.DMA,
        pltpu.VMEM((TILE,), jnp.int32),
        pltpu.SemaphoreType.DMA,
    ]

    @functools.partial(
        pl.kernel,
        out_type=[jax.ShapeDtypeStruct((n_pad, HALF), jnp.float32),
                  jax.ShapeDtypeStruct((n_pad, HALF), jnp.float32),
                  jax.ShapeDtypeStruct((n_pad, NARROW), jnp.float32),
                  jax.ShapeDtypeStruct((n_pad, NARROW), jnp.float32)],
        mesh=mesh,
        compiler_params=cp,
        scratch_types=bufset + bufset + [
            pltpu.VMEM_SHARED((n_pad, HALF), jnp.float32),
            pltpu.VMEM_SHARED((n_pad, NARROW), jnp.float32),
            pltpu.SemaphoreType.DMA,
        ],
    )
    def body(src_hbm, dst_hbm, aug0_hbm, aug1_hbm, nar_hbm, zrow_hbm,
             zrow8_hbm, out0_hbm, out1_hbm, outn0_hbm, outn1_hbm, *refs):
        bufs = (refs[0:9], refs[9:18])
        acc, accn = refs[18], refs[19]
        zsem = refs[20]
        cid = lax.axis_index("c")
        sid = lax.axis_index("s")
        nk = (n_tiles - sid + NS - 1) // NS

        def idx_copies(k, buf):
            t = (sid + k * NS) * TILE
            return [pltpu.make_async_copy(src_hbm.at[pl.ds(t, TILE)], buf[0],
                                          buf[6]),
                    pltpu.make_async_copy(dst_hbm.at[pl.ds(t, TILE)], buf[1],
                                          buf[6])]

        def narrow_here(k):
            return (k % 2) == cid

        def gather_copies(buf):
            srcv, dstv = buf[0], buf[1]
            g0 = [pltpu.make_async_copy(aug0_hbm.at[srcv], buf[2], buf[5])]
            g1 = [pltpu.make_async_copy(aug1_hbm.at[srcv], buf[2], buf[5])]
            gn = [pltpu.make_async_copy(nar_hbm.at[srcv], buf[3], buf[5]),
                  pltpu.make_async_copy(nar_hbm.at[dstv], buf[4], buf[5])]
            return g0, g1, gn

        def start_gathers(k, buf):
            g0, g1, gn = gather_copies(buf)

            @pl.when(cid == 0)
            def _():
                g0[0].start()

            @pl.when(cid == 1)
            def _():
                g1[0].start()

            @pl.when(narrow_here(k))
            def _():
                for c in gn:
                    c.start()

        def wait_gathers(k, buf):
            g0, g1, gn = gather_copies(buf)

            @pl.when(cid == 0)
            def _():
                g0[0].wait()

            @pl.when(cid == 1)
            def _():
                g1[0].wait()

            @pl.when(narrow_here(k))
            def _():
                for c in gn:
                    c.wait()

        def narrow_work(k, buf):
            @pl.when(narrow_here(k))
            def _():
                ns, nd = buf[3], buf[4]

                @pl.loop(0, TILE // LANES)
                def _(i):
                    rowid = lax.iota(jnp.int32, LANES) + i * LANES
                    c0 = jnp.full((LANES,), 0, jnp.int32)
                    c1 = jnp.full((LANES,), 1, jnp.int32)
                    c2 = jnp.full((LANES,), 2, jnp.int32)
                    dx = (plsc.load_gather(nd, [rowid, c0])
                          - plsc.load_gather(ns, [rowid, c0]))
                    dy = (plsc.load_gather(nd, [rowid, c1])
                          - plsc.load_gather(ns, [rowid, c1]))
                    dz = (plsc.load_gather(nd, [rowid, c2])
                          - plsc.load_gather(ns, [rowid, c2]))
                    d2 = dx * dx + dy * dy + dz * dz
                    d2c = jnp.maximum(d2, 1e-30)
                    bits = plsc.bitcast(d2c, jnp.int32)
                    y = plsc.bitcast(jnp.int32(0x5F3759DF) - (bits >> 1),
                                     jnp.float32)
                    y = y * (1.5 - 0.5 * d2c * y * y)
                    y = y * (1.5 - 0.5 * d2c * y * y)
                    y = y * (1.5 - 0.5 * d2c * y * y)
                    dist = d2 * y
                    cd = jnp.full((LANES,), NDIST, jnp.int32)
                    plsc.store_scatter(ns, [rowid, cd], dist)

        def start_scatters(k, buf):
            @pl.loop(0, TILE // LANES)
            def _(i):
                sl = pl.ds(i * LANES, LANES)
                buf[7][sl] = buf[1][sl]
            pltpu.async_copy(buf[2], acc.at[buf[7]], buf[8], add=True)

            @pl.when(narrow_here(k))
            def _():
                pltpu.async_copy(buf[3], accn.at[buf[7]], buf[8], add=True)

        def wait_scatters(k, buf):
            pltpu.make_async_copy(buf[2], acc.at[buf[7]], buf[8]).wait()

            @pl.when(narrow_here(k))
            def _():
                pltpu.make_async_copy(buf[3], accn.at[buf[7]], buf[8]).wait()

        def process(k, cur, nxt):
            @pl.when(k + 1 < nk)
            def _():
                @pl.when(k >= 1)
                def _():
                    wait_scatters(k - 1, nxt)
                for c in idx_copies(k + 1, nxt):
                    c.wait()
                start_gathers(k + 1, nxt)
            wait_gathers(k, cur)
            narrow_work(k, cur)
            start_scatters(k, cur)

            @pl.when(k + 2 < nk)
            def _():
                for c in idx_copies(k + 2, cur):
                    c.start()

        pltpu.sync_copy(zrow_hbm, bufs[0][2])
        pltpu.sync_copy(zrow8_hbm, bufs[0][3])
        jmax = (nblocks + NS - 1) // NS
        for j in range(jmax):
            b = (sid + j * NS) * TILE

            @pl.when(sid + j * NS < nblocks)
            def _():
                pltpu.async_copy(bufs[0][2], acc.at[pl.ds(b, TILE)], zsem)
                pltpu.async_copy(bufs[0][3], accn.at[pl.ds(b, TILE)], zsem)
        for j in range(jmax):
            b = (sid + j * NS) * TILE

            @pl.when(sid + j * NS < nblocks)
            def _():
                pltpu.make_async_copy(bufs[0][2], acc.at[pl.ds(b, TILE)],
                                      zsem).wait()
                pltpu.make_async_copy(bufs[0][3], accn.at[pl.ds(b, TILE)],
                                      zsem).wait()

        plsc.subcore_barrier()

        @pl.when(nk > 0)
        def _():
            cs = idx_copies(0, bufs[0])
            for c in cs:
                c.start()
            for c in cs:
                c.wait()
            start_gathers(0, bufs[0])

            @pl.when(1 < nk)
            def _():
                for c in idx_copies(1, bufs[1]):
                    c.start()

        @pl.loop(0, (nk + 1) // 2)
        def _(p):
            process(2 * p, bufs[0], bufs[1])

            @pl.when(2 * p + 1 < nk)
            def _():
                process(2 * p + 1, bufs[1], bufs[0])

        @pl.when(nk >= 1)
        def _():
            wait_scatters(0, bufs[0])

        @pl.when(nk >= 2)
        def _():
            wait_scatters(1, bufs[1])

        plsc.subcore_barrier()

        def wb(out_hbm, outn_hbm, issue):
            for j in range(jmax):
                b = (sid + j * NS) * TILE

                @pl.when(sid + j * NS < nblocks)
                def _():
                    c1 = pltpu.make_async_copy(
                        acc.at[pl.ds(b, TILE)], out_hbm.at[pl.ds(b, TILE)],
                        zsem)
                    c2 = pltpu.make_async_copy(
                        accn.at[pl.ds(b, TILE)], outn_hbm.at[pl.ds(b, TILE)],
                        zsem)
                    if issue:
                        c1.start()
                        c2.start()
                    else:
                        c1.wait()
                        c2.wait()

        @pl.when(cid == 0)
        def _():
            wb(out0_hbm, outn0_hbm, True)
            wb(out0_hbm, outn0_hbm, False)

        @pl.when(cid == 1)
        def _():
            wb(out1_hbm, outn1_hbm, True)
            wb(out1_hbm, outn1_hbm, False)

    return body(src, dst, aug0, aug1, nar, zrow, zrow8)


def _tc_fused_out(acc0, acc1, accn0, accn1, feat, pos, w0, w1, ws, small,
                  n, d_out):
    block_m = 1000
    hi = lax.Precision.DEFAULT

    def mm(a0_ref, a1_ref, n0_ref, n1_ref, f_ref, p_ref, w0_ref, w1_ref,
           ws_ref, s_ref, o_ref):
        nb = n0_ref[...] + n1_ref[...]
        deg = nb[:, NONEC:NONEC + 1]
        s = s_ref[...]
        o = jnp.dot(a0_ref[...], w0_ref[...],
                    preferred_element_type=jnp.float32, precision=hi)
        o += jnp.dot(a1_ref[...], w1_ref[...],
                     preferred_element_type=jnp.float32, precision=hi)
        o += jnp.dot(deg * f_ref[...], ws_ref[...],
                     preferred_element_type=jnp.float32, precision=hi)
        p = p_ref[...]
        for c in range(3):
            rel = deg * p[:, c:c + 1] - nb[:, c:c + 1]
            o += rel * s[c:c + 1, :]
        o += nb[:, NDIST:NDIST + 1] * s[3:4, :]
        o += deg * s[4:5, :]
        o_ref[...] = o

    return pl.pallas_call(
        mm,
        grid=(n // block_m,),
        in_specs=[pl.BlockSpec((block_m, HALF), lambda i: (i, 0)),
                  pl.BlockSpec((block_m, HALF), lambda i: (i, 0)),
                  pl.BlockSpec((block_m, NARROW), lambda i: (i, 0)),
                  pl.BlockSpec((block_m, NARROW), lambda i: (i, 0)),
                  pl.BlockSpec((block_m, 256), lambda i: (i, 0)),
                  pl.BlockSpec((block_m, 3), lambda i: (i, 0)),
                  pl.BlockSpec((HALF, 256), lambda i: (0, 0)),
                  pl.BlockSpec((HALF, 256), lambda i: (0, 0)),
                  pl.BlockSpec((256, 256), lambda i: (0, 0)),
                  pl.BlockSpec((8, 256), lambda i: (0, 0))],
        out_specs=pl.BlockSpec((block_m, d_out), lambda i: (i, 0)),
        out_shape=jax.ShapeDtypeStruct((n, d_out), jnp.float32),
    )(acc0, acc1, accn0, accn1, feat, pos, w0, w1, ws, small)


def kernel(input_feature, pos, edge_index, W_neighbor, b_neighbor, W_self,
           b_self):
    n, d_in = input_feature.shape
    e = edge_index.shape[1]
    d_out = W_self.shape[1]
    assert e % TILE == 0 and d_in == 2 * HALF
    n_tiles = e // TILE
    n_pad = ((n + TILE - 1) // TILE) * TILE

    feat = input_feature.astype(jnp.float32)
    pos = pos.astype(jnp.float32)
    src = edge_index[0].astype(jnp.int32)
    dst = edge_index[1].astype(jnp.int32)

    aug0 = feat[:, :HALF]
    aug1 = feat[:, HALF:]
    nar = jnp.concatenate(
        [pos, jnp.ones((n, 1), jnp.float32),
         jnp.zeros((n, NARROW - NONEC - 1), jnp.float32)], axis=1)
    zrow = jnp.zeros((TILE, HALF), jnp.float32)
    zrow8 = jnp.zeros((TILE, NARROW), jnp.float32)

    acc0, acc1, accn0, accn1 = _sc_segment_sums(
        src, dst, aug0, aug1, nar, zrow, zrow8, n_pad, n_tiles)

    w0 = W_neighbor[:HALF]
    w1 = W_neighbor[HALF:d_in]
    small = jnp.concatenate(
        [W_neighbor[d_in:d_in + 4], (b_neighbor + b_self)[None],
         jnp.zeros((3, d_out), jnp.float32)], axis=0)

    return _tc_fused_out(acc0, acc1, accn0, accn1, feat, pos, w0, w1, W_self,
                         small, n, d_out)
